# baseline (speedup 1.0000x reference)
import functools

import jax
import jax.numpy as jnp
from jax import lax
from jax.experimental import pallas as pl
from jax.experimental.pallas import tpu as pltpu
from jax.experimental.pallas import tpu_sc as plsc

T_OUT = 2048
_C = 32


@functools.lru_cache(maxsize=None)
def _build_sc_kernel(N, P, D, NW, NC):
    rows_w = N // NW
    nchunks = rows_w // _C
    pad_w = P // NW
    pchunks = pad_w // _C

    mesh = plsc.VectorSubcoreMesh(core_axis_name="c", subcore_axis_name="s")

    @functools.partial(
        pl.kernel,
        mesh=mesh,
        out_type=jax.ShapeDtypeStruct((N + P, D), jnp.float32),
        scratch_types=[
            pltpu.VMEM((nchunks, _C), jnp.int32),
            pltpu.VMEM((pchunks, _C), jnp.int32),
            pltpu.VMEM((_C, D), jnp.float32),
            pltpu.VMEM((_C, D), jnp.float32),
            pltpu.SemaphoreType.DMA,
        ],
    )
    def k(x_hbm, sidx_hbm, zidx_hbm, zsrc_hbm, out_hbm,
          sidx_v, zidx_v, rows_v, zero_v, sem):
        wid = lax.axis_index("s") * NC + lax.axis_index("c")
        pltpu.sync_copy(sidx_hbm.at[wid], sidx_v)
        pltpu.sync_copy(zidx_hbm.at[wid], zidx_v)
        pltpu.sync_copy(zsrc_hbm, zero_v)
        base = wid * rows_w

        def scat(i, carry):
            pltpu.sync_copy(x_hbm.at[pl.ds(base + i * _C, _C), :], rows_v)
            pltpu.async_copy(rows_v, out_hbm.at[sidx_v.at[i]], sem).wait()
            return carry

        lax.fori_loop(0, nchunks, scat, 0)

        def zfill(i, carry):
            pltpu.async_copy(zero_v, out_hbm.at[zidx_v.at[i]], sem).wait()
            return carry

        lax.fori_loop(0, pchunks, zfill, 0)

    return k


def kernel(x, lengths):
    N, D = x.shape
    B = lengths.shape[0]
    T = T_OUT
    P = B * T - N

    info = plsc.get_sparse_core_info()
    NC, NS = info.num_cores, info.num_subcores
    NW = NC * NS

    t = jnp.arange(T, dtype=jnp.int32)
    bs = jnp.sum(lengths[None, :] > t[:, None], axis=1).astype(jnp.int32)
    prefix = jnp.concatenate([jnp.zeros((1,), jnp.int32),
                              jnp.cumsum(bs)[:-1].astype(jnp.int32)])
    b = jnp.arange(B, dtype=jnp.int32)[:, None]
    idx = prefix[None, :] + b
    mask = t[None, :] < lengths[:, None]
    dest = b * T + t[None, :]

    sidx = jnp.zeros((N,), jnp.int32).at[jnp.where(mask, idx, N)].set(
        dest, mode="drop")
    flat_mask = mask.reshape(-1)
    rank = jnp.cumsum(~flat_mask).astype(jnp.int32) - 1
    zidx = jnp.zeros((P,), jnp.int32).at[jnp.where(flat_mask, P, rank)].set(
        jnp.arange(B * T, dtype=jnp.int32), mode="drop")

    zsrc = jnp.zeros((_C, D), x.dtype)
    k = _build_sc_kernel(N, P, D, NW, NC)
    out = k(x, sidx.reshape(NW, -1, _C), zidx.reshape(NW, -1, _C), zsrc)
    return out.reshape(B, T, D)


# speedup vs baseline: 1.0141x; 1.0141x over previous
import functools

import jax
import jax.numpy as jnp
from jax import lax
from jax.experimental import pallas as pl
from jax.experimental.pallas import tpu as pltpu
from jax.experimental.pallas import tpu_sc as plsc

T_OUT = 2048
_C = 32
_CZ = 16
_NB = 3


@functools.lru_cache(maxsize=None)
def _build_sc_kernel(N, P, D, NW, NC):
    rows_w = N // NW
    nchunks = rows_w // _C
    pad_w = P // NW
    pchunks = pad_w // _CZ

    mesh = plsc.VectorSubcoreMesh(core_axis_name="c", subcore_axis_name="s")

    @functools.partial(
        pl.kernel,
        mesh=mesh,
        out_type=jax.ShapeDtypeStruct((N + P, D), jnp.float32),
        scratch_types=[
            pltpu.VMEM((nchunks, _C), jnp.int32),
            pltpu.VMEM((pchunks, _CZ), jnp.int32),
            pltpu.VMEM((_C, D), jnp.float32),
            pltpu.VMEM((_C, D), jnp.float32),
            pltpu.VMEM((_C, D), jnp.float32),
            pltpu.VMEM((_CZ, D), jnp.float32),
            pltpu.SemaphoreType.DMA,
            pltpu.SemaphoreType.DMA,
            pltpu.SemaphoreType.DMA,
            pltpu.SemaphoreType.DMA,
            pltpu.SemaphoreType.DMA,
            pltpu.SemaphoreType.DMA,
            pltpu.SemaphoreType.DMA,
        ],
    )
    def k(x_hbm, sidx_hbm, zidx_hbm, zsrc_hbm, out_hbm,
          sidx_v, zidx_v, buf0, buf1, buf2, zero_v,
          l0, l1, l2, s0, s1, s2, zsem):
        bufs = (buf0, buf1, buf2)
        lsem = (l0, l1, l2)
        ssem = (s0, s1, s2)
        wid = lax.axis_index("s") * NC + lax.axis_index("c")
        pltpu.sync_copy(sidx_hbm.at[wid], sidx_v)
        pltpu.sync_copy(zidx_hbm.at[wid], zidx_v)
        pltpu.sync_copy(zsrc_hbm, zero_v)
        base = wid * rows_w

        def load(i, b, sem_i):
            return pltpu.make_async_copy(
                x_hbm.at[pl.ds(base + i * _C, _C), :], bufs[b], lsem[sem_i])

        def scat(i, b, sem_i):
            return pltpu.make_async_copy(
                bufs[b], out_hbm.at[sidx_v.at[i]], ssem[sem_i])

        def zfire(j, carry):
            pltpu.async_copy(zero_v, out_hbm.at[zidx_v.at[j]], zsem)
            return carry

        lax.fori_loop(0, pchunks, zfire, 0)

        load(0, 0, 0).start()

        def body(i, carry):
            for b in range(_NB):
                c = (b + 1) % _NB

                @pl.when(i % _NB == b)
                def _(b=b, c=c):
                    load(i, b, b).wait()
                    scat(i, b, b).start()

                    @pl.when(i + 1 < nchunks)
                    def _(b=b, c=c):
                        @pl.when(i >= _NB - 1)
                        def _(c=c):
                            scat(i - (_NB - 1), c, c).wait()

                        load(i + 1, c, c).start()

            return carry

        lax.fori_loop(0, nchunks, body, 0)

        for j in range(nchunks - _NB, nchunks):
            scat(j, j % _NB, j % _NB).wait()

        def zdrain(j, carry):
            pltpu.make_async_copy(zero_v, out_hbm.at[zidx_v.at[j]],
                                  zsem).wait()
            return carry

        lax.fori_loop(0, pchunks, zdrain, 0)

    return k


def kernel(x, lengths):
    N, D = x.shape
    B = lengths.shape[0]
    T = T_OUT
    P = B * T - N

    info = plsc.get_sparse_core_info()
    NC, NS = info.num_cores, info.num_subcores
    NW = NC * NS

    t = jnp.arange(T, dtype=jnp.int32)
    bs = jnp.sum(lengths[None, :] > t[:, None], axis=1).astype(jnp.int32)
    prefix = jnp.concatenate([jnp.zeros((1,), jnp.int32),
                              jnp.cumsum(bs)[:-1].astype(jnp.int32)])
    b = jnp.arange(B, dtype=jnp.int32)[:, None]
    idx = prefix[None, :] + b
    mask = t[None, :] < lengths[:, None]
    dest = b * T + t[None, :]

    sidx = jnp.zeros((N,), jnp.int32).at[jnp.where(mask, idx, N)].set(
        dest, mode="drop")
    flat_mask = mask.reshape(-1)
    rank = jnp.cumsum(~flat_mask).astype(jnp.int32) - 1
    zidx = jnp.zeros((P,), jnp.int32).at[jnp.where(flat_mask, P, rank)].set(
        jnp.arange(B * T, dtype=jnp.int32), mode="drop")

    zsrc = jnp.zeros((_CZ, D), x.dtype)
    k = _build_sc_kernel(N, P, D, NW, NC)
    out = k(x, sidx.reshape(NW, -1, _C), zidx.reshape(NW, -1, _CZ), zsrc)
    return out.reshape(B, T, D)
